# baseline (speedup 1.0000x reference)
import jax
import jax.numpy as jnp
from jax.experimental import pallas as pl
from jax.experimental.pallas import tpu as pltpu


def _fused_kernel(x_ref, hg_ref, w1_ref, b1_ref, w2_ref, b2_ref, out_ref,
                  acc_ref, de_ref, y_ref):
    s = pl.program_id(0)
    n = pl.program_id(1)
    nsteps = pl.num_programs(1)

    @pl.when(s == 0)
    def _v2e():
        h = hg_ref[...]
        x1 = jnp.dot(x_ref[...], w1_ref[...],
                     preferred_element_type=jnp.float32) + b1_ref[...]
        part = jax.lax.dot_general(
            h, x1.astype(jnp.bfloat16),
            (((0,), (0,)), ((), ())),
            preferred_element_type=jnp.float32)
        de_part = jnp.sum(h, axis=0, keepdims=True,
                          dtype=jnp.float32)

        @pl.when(n == 0)
        def _init():
            acc_ref[...] = part
            de_ref[...] = de_part

        @pl.when(n > 0)
        def _acc():
            acc_ref[...] += part
            de_ref[...] += de_part

        @pl.when(n == nsteps - 1)
        def _finish():
            de = de_ref[...]
            scale = jnp.transpose(
                jnp.where(de > 0.0, 1.0 / de, 0.0))
            y = jnp.maximum(acc_ref[...] * scale, 0.0)
            y2 = jnp.dot(y.astype(jnp.bfloat16),
                         w2_ref[...].astype(jnp.bfloat16),
                         preferred_element_type=jnp.float32) + b2_ref[...]
            y_ref[...] = y2.astype(jnp.bfloat16)

    @pl.when(s == 1)
    def _e2v():
        h = hg_ref[...]
        xv = jnp.dot(h, y_ref[...],
                     preferred_element_type=jnp.float32)
        dv = jnp.sum(h, axis=1, keepdims=True, dtype=jnp.float32)
        scale = jnp.where(dv > 0.0, 1.0 / dv, 0.0)
        out_ref[...] = jnp.maximum(xv * scale, 0.0)


@jax.jit
def kernel(X, hg, W_v2e, b_v2e, W_e2v, b_e2v):
    N, C = X.shape
    M = hg.shape[1]
    lanes = 128
    Mp = ((M + lanes - 1) // lanes) * lanes
    blk = 1000
    assert N % blk == 0

    hgp = jnp.pad(hg.astype(jnp.bfloat16), ((0, 0), (0, Mp - M)))

    b1 = b_v2e.reshape(1, C)
    b2 = b_e2v.reshape(1, C)

    xv = pl.pallas_call(
        _fused_kernel,
        grid=(2, N // blk),
        in_specs=[
            pl.BlockSpec((blk, C), lambda s, n: (n, 0)),
            pl.BlockSpec((blk, Mp), lambda s, n: (n, 0)),
            pl.BlockSpec((C, C), lambda s, n: (0, 0)),
            pl.BlockSpec((1, C), lambda s, n: (0, 0)),
            pl.BlockSpec((C, C), lambda s, n: (0, 0)),
            pl.BlockSpec((1, C), lambda s, n: (0, 0)),
        ],
        out_specs=pl.BlockSpec((blk, C), lambda s, n: (n, 0)),
        out_shape=jax.ShapeDtypeStruct((N, C), jnp.float32),
        scratch_shapes=[
            pltpu.VMEM((Mp, C), jnp.float32),
            pltpu.VMEM((1, Mp), jnp.float32),
            pltpu.VMEM((Mp, C), jnp.bfloat16),
        ],
        compiler_params=pltpu.CompilerParams(
            dimension_semantics=("arbitrary", "arbitrary"),
            fuse_transposed_lhs_in_matmul=True),
    )(X, hgp, W_v2e, b1, W_e2v, b2)

    return xv


# speedup vs baseline: 2.1534x; 2.1534x over previous
import jax
import jax.numpy as jnp
from jax.experimental import pallas as pl
from jax.experimental.pallas import tpu as pltpu


def _pick_block(n, target=1000):
    best = None
    for b in range(8, target + 1, 8):
        if n % b == 0:
            best = b
    return best


def _fused_kernel(x_ref, hg_ref, w1_ref, b1_ref, w2_ref, b2_ref, out_ref,
                  acc_ref, y_ref):
    s = pl.program_id(0)
    n = pl.program_id(1)
    nsteps = pl.num_programs(1)
    C = w1_ref.shape[0]
    blk = x_ref.shape[0]

    @pl.when(s == 0)
    def _v2e():
        h16 = hg_ref[...].astype(jnp.bfloat16)
        x1 = jnp.dot(x_ref[...], w1_ref[...],
                     preferred_element_type=jnp.float32) + b1_ref[...]
        x1aug = jnp.concatenate(
            [x1, jnp.ones((blk, 8), jnp.float32)], axis=1).astype(jnp.bfloat16)
        part = jax.lax.dot_general(
            x1aug, h16, (((0,), (0,)), ((), ())),
            preferred_element_type=jnp.float32)

        @pl.when(n == 0)
        def _init():
            acc_ref[...] = part

        @pl.when(n > 0)
        def _acc():
            acc_ref[...] += part

        @pl.when(n == nsteps - 1)
        def _finish():
            de = acc_ref[C:C + 1, :]
            y = jnp.maximum(acc_ref[:C, :] * (1.0 / de), 0.0)
            y2 = jax.lax.dot_general(
                w2_ref[...].astype(jnp.bfloat16), y.astype(jnp.bfloat16),
                (((0,), (0,)), ((), ())),
                preferred_element_type=jnp.float32) + b2_ref[...]
            y_ref[...] = jnp.transpose(y2).astype(jnp.bfloat16)

    @pl.when(s == 1)
    def _e2v():
        h = hg_ref[...]
        xv = jnp.dot(h.astype(jnp.bfloat16), y_ref[...],
                     preferred_element_type=jnp.float32)
        dv = jnp.sum(h, axis=1, keepdims=True)
        scale = jnp.where(dv > 0.0, 1.0 / dv, 0.0)
        out_ref[...] = jnp.maximum(xv * scale, 0.0)


@jax.jit
def kernel(X, hg, W_v2e, b_v2e, W_e2v, b_e2v):
    N, C = X.shape
    M = hg.shape[1]
    blk = _pick_block(N)

    b1 = b_v2e.reshape(1, C)
    b2 = b_e2v.reshape(C, 1)

    xv = pl.pallas_call(
        _fused_kernel,
        grid=(2, N // blk),
        in_specs=[
            pl.BlockSpec((blk, C), lambda s, n: (n, 0)),
            pl.BlockSpec((blk, M), lambda s, n: (n, 0)),
            pl.BlockSpec((C, C), lambda s, n: (0, 0)),
            pl.BlockSpec((1, C), lambda s, n: (0, 0)),
            pl.BlockSpec((C, C), lambda s, n: (0, 0)),
            pl.BlockSpec((C, 1), lambda s, n: (0, 0)),
        ],
        out_specs=pl.BlockSpec((blk, C), lambda s, n: (n, 0)),
        out_shape=jax.ShapeDtypeStruct((N, C), jnp.float32),
        scratch_shapes=[
            pltpu.VMEM((C + 8, M), jnp.float32),
            pltpu.VMEM((M, C), jnp.bfloat16),
        ],
        compiler_params=pltpu.CompilerParams(
            dimension_semantics=("arbitrary", "arbitrary")),
    )(X, hg, W_v2e, b1, W_e2v, b2)

    return xv


# speedup vs baseline: 2.1604x; 1.0032x over previous
import jax
import jax.numpy as jnp
from jax.experimental import pallas as pl
from jax.experimental.pallas import tpu as pltpu


def _pick_block(n, target=1000):
    best = None
    for b in range(8, target + 1, 8):
        if n % b == 0:
            best = b
    return best


def _fused_kernel(x_ref, hg_ref, w1_ref, b1_ref, w2_ref, b2_ref, out_ref,
                  acc_ref, y_ref):
    s = pl.program_id(0)
    n = pl.program_id(1)
    nsteps = pl.num_programs(1)
    C = w1_ref.shape[0]
    blk = x_ref.shape[0]

    @pl.when(s == 0)
    def _v2e():
        h16 = hg_ref[...].astype(jnp.bfloat16)
        x1 = jnp.dot(x_ref[...], w1_ref[...],
                     preferred_element_type=jnp.float32) + b1_ref[...]
        x1aug = jnp.concatenate(
            [x1, jnp.ones((blk, 8), jnp.float32)], axis=1).astype(jnp.bfloat16)
        part = jax.lax.dot_general(
            x1aug, h16, (((0,), (0,)), ((), ())),
            preferred_element_type=jnp.float32)

        @pl.when(n == 0)
        def _init():
            acc_ref[...] = part

        @pl.when(n > 0)
        def _acc():
            acc_ref[...] += part

        @pl.when(n == nsteps - 1)
        def _finish():
            de = acc_ref[C:C + 1, :]
            y = jnp.maximum(acc_ref[:C, :] * (1.0 / de), 0.0)
            y2 = jax.lax.dot_general(
                w2_ref[...].astype(jnp.bfloat16), y.astype(jnp.bfloat16),
                (((0,), (0,)), ((), ())),
                preferred_element_type=jnp.float32) + b2_ref[...]
            y_ref[...] = jnp.transpose(y2).astype(jnp.bfloat16)

    @pl.when(s == 1)
    def _e2v():
        h = hg_ref[...]
        xv = jnp.dot(h.astype(jnp.bfloat16), y_ref[...],
                     preferred_element_type=jnp.float32)
        dv = jnp.sum(h, axis=1, keepdims=True)
        scale = jnp.where(dv > 0.0, 1.0 / dv, 0.0)
        out_ref[...] = jnp.maximum(xv * scale, 0.0)


@jax.jit
def kernel(X, hg, W_v2e, b_v2e, W_e2v, b_e2v):
    N, C = X.shape
    M = hg.shape[1]
    blk = _pick_block(N)

    b1 = b_v2e.reshape(1, C)
    b2 = b_e2v.reshape(C, 1)

    xv = pl.pallas_call(
        _fused_kernel,
        grid=(2, N // blk),
        in_specs=[
            pl.BlockSpec((blk, C), lambda s, n: (n, 0)),
            pl.BlockSpec((blk, M), lambda s, n: (n, 0)),
            pl.BlockSpec((C, C), lambda s, n: (0, 0)),
            pl.BlockSpec((1, C), lambda s, n: (0, 0)),
            pl.BlockSpec((C, C), lambda s, n: (0, 0)),
            pl.BlockSpec((C, 1), lambda s, n: (0, 0)),
        ],
        out_specs=pl.BlockSpec((blk, C), lambda s, n: (n, 0)),
        out_shape=jax.ShapeDtypeStruct((N, C), jnp.float32),
        scratch_shapes=[
            pltpu.VMEM((C + 8, M), jnp.float32),
            pltpu.VMEM((M, C), jnp.bfloat16),
        ],
        compiler_params=pltpu.CompilerParams(
            dimension_semantics=("arbitrary", "arbitrary"),
            allow_input_fusion=[False, True, False, False, False, False]),
    )(X, hg, W_v2e, b1, W_e2v, b2)

    return xv
